# Initial kernel scaffold; baseline (speedup 1.0000x reference)
import jax
import jax.numpy as jnp
from jax.experimental import pallas as pl


def kernel(xyz, points):
    raise NotImplementedError("write your pallas kernel here")


# speedup vs baseline: 4.5686x; 4.5686x over previous
import functools

import jax
import jax.numpy as jnp
from jax import lax
from jax.experimental import pallas as pl
from jax.experimental.pallas import tpu as pltpu
from jax.experimental.pallas import tpu_sc as plsc

B = 4
N = 2048
S = N // 2
CP = 128
SUB = 16
BLK = 128
MIN_R2 = 0.02 ** 2
MAX_R2 = 0.05 ** 2


def _r2max(x):
    return jnp.max(jnp.max(x, axis=2, keepdims=True), axis=1, keepdims=True)


def _r2min(x):
    return jnp.min(jnp.min(x, axis=2, keepdims=True), axis=1, keepdims=True)


def _r2sum(x):
    return jnp.sum(jnp.sum(x, axis=2, keepdims=True), axis=1, keepdims=True)


def _tc_body(xyzr_ref, out_ref, cenx_s, ceny_s, cenz_s, fps_s):
    X = xyzr_ref[:, 0]
    Y = xyzr_ref[:, 1]
    Z = xyzr_ref[:, 2]
    iota3 = (lax.broadcasted_iota(jnp.int32, (B, SUB, 128), 1) * 128
             + lax.broadcasted_iota(jnp.int32, (B, SUB, 128), 2))
    lane = lax.broadcasted_iota(jnp.int32, (B, BLK), 1)

    def step(i, carry):
        D, far, bfar, bcx, bcy, bcz = carry
        sel = iota3 == far
        cx = _r2sum(jnp.where(sel, X, 0.0))
        cy = _r2sum(jnp.where(sel, Y, 0.0))
        cz = _r2sum(jnp.where(sel, Z, 0.0))
        dx = X - cx
        dy = Y - cy
        dz = Z - cz
        dist = (dx * dx + dy * dy) + dz * dz
        D = jnp.minimum(D, dist)
        m = _r2max(D)
        far_new = _r2min(jnp.where(D == m, iota3, N))
        put = lane == i
        bfar = jnp.where(put, far[:, :, 0], bfar)
        bcx = jnp.where(put, cx[:, :, 0], bcx)
        bcy = jnp.where(put, cy[:, :, 0], bcy)
        bcz = jnp.where(put, cz[:, :, 0], bcz)
        return D, far_new, bfar, bcx, bcy, bcz

    nblk = S // BLK

    def fps_blk(blk, carry):
        D, far = carry
        bfar = jnp.zeros((B, BLK), jnp.int32)
        bcx = jnp.zeros((B, BLK), jnp.float32)
        bcy = jnp.zeros((B, BLK), jnp.float32)
        bcz = jnp.zeros((B, BLK), jnp.float32)
        D, far, bfar, bcx, bcy, bcz = lax.fori_loop(
            0, BLK, step, (D, far, bfar, bcx, bcy, bcz))
        fps_s[:, pl.ds(blk, 1), :] = bfar.reshape(B, 1, BLK)
        cenx_s[:, pl.ds(blk, 1), :] = bcx.reshape(B, 1, BLK)
        ceny_s[:, pl.ds(blk, 1), :] = bcy.reshape(B, 1, BLK)
        cenz_s[:, pl.ds(blk, 1), :] = bcz.reshape(B, 1, BLK)
        return D, far

    D0 = jnp.full((B, SUB, 128), 1e10, jnp.float32)
    far0 = jnp.zeros((B, 1, 1), jnp.int32)
    lax.fori_loop(0, nblk, fps_blk, (D0, far0))

    SN = (X * X + Y * Y) + Z * Z
    Xb = X.astype(jnp.bfloat16).astype(jnp.float32)
    Yb = Y.astype(jnp.bfloat16).astype(jnp.float32)
    Zb = Z.astype(jnp.bfloat16).astype(jnp.float32)
    boffs = lax.broadcasted_iota(jnp.int32, (B, 1, 1), 0) * N
    sub8 = lax.broadcasted_iota(jnp.int32, (B, nblk, BLK), 1)
    lane8 = lax.broadcasted_iota(jnp.int32, (B, nblk, BLK), 2)
    cenx_all = cenx_s[:, :, :]
    ceny_all = ceny_s[:, :, :]
    cenz_all = cenz_s[:, :, :]
    fps_all = fps_s[:, :, :]

    def q_blk(blk2, _):
        def q_one(j, brow):
            pick = (sub8 == blk2) & (lane8 == j)
            cx = _r2sum(jnp.where(pick, cenx_all, 0.0))
            cy = _r2sum(jnp.where(pick, ceny_all, 0.0))
            cz = _r2sum(jnp.where(pick, cenz_all, 0.0))
            fpsr = _r2sum(jnp.where(pick, fps_all, 0))
            sc = (cx * cx + cy * cy) + cz * cz
            cxb = cx.astype(jnp.bfloat16).astype(jnp.float32)
            cyb = cy.astype(jnp.bfloat16).astype(jnp.float32)
            czb = cz.astype(jnp.bfloat16).astype(jnp.float32)
            E = (Xb * cxb + Yb * cyb) + Zb * czb
            d = (-2.0 * E + sc) + SN
            mask = (d > MIN_R2) & (d < MAX_R2)
            cand = jnp.where(mask, d, jnp.inf)
            mn = _r2min(cand)
            near = _r2min(jnp.where(cand == mn, iota3, N))
            has = mn < jnp.inf
            row = jnp.where(has, near, fpsr) + boffs
            return jnp.where(lane == j, row[:, :, 0], brow)

        brow = lax.fori_loop(0, BLK, q_one, jnp.zeros((B, BLK), jnp.int32))
        out_ref[:, pl.ds(blk2, 1), :] = brow.reshape(B, 1, BLK)
        return 0

    lax.fori_loop(0, nblk, q_blk, 0)


@jax.jit
def _tc_indices(xyzr):
    return pl.pallas_call(
        _tc_body,
        out_shape=jax.ShapeDtypeStruct((B, S // BLK, BLK), jnp.int32),
        scratch_shapes=[
            pltpu.VMEM((B, S // BLK, BLK), jnp.float32),
            pltpu.VMEM((B, S // BLK, BLK), jnp.float32),
            pltpu.VMEM((B, S // BLK, BLK), jnp.float32),
            pltpu.VMEM((B, S // BLK, BLK), jnp.int32),
        ],
    )(xyzr)


@jax.jit
def _sc_gather(ptab, xtab, idx):
    info = plsc.get_sparse_core_info()
    nc, ns = info.num_cores, info.num_subcores
    nw = nc * ns
    bpw = (B * S) // nw
    mesh = plsc.VectorSubcoreMesh(core_axis_name="c", subcore_axis_name="s")

    @functools.partial(
        pl.kernel,
        out_type=(jax.ShapeDtypeStruct((B * S, CP), jnp.float32),
                  jax.ShapeDtypeStruct((B * S, 128), jnp.float32)),
        mesh=mesh,
        scratch_types=[
            pltpu.VMEM((bpw,), jnp.int32),
            pltpu.VMEM((bpw, CP), jnp.float32),
            pltpu.VMEM((bpw, 128), jnp.float32),
            pltpu.SemaphoreType.DMA,
        ],
    )
    def k(ptab_hbm, xtab_hbm, idx_hbm, outp_hbm, outx_hbm,
          idx_v, prow_v, xrow_v, sem):
        wid = lax.axis_index("s") * nc + lax.axis_index("c")
        base = wid * bpw
        pltpu.sync_copy(idx_hbm.at[pl.ds(base, bpw)], idx_v)
        pltpu.async_copy(ptab_hbm.at[idx_v], prow_v, sem).wait()
        pltpu.async_copy(xtab_hbm.at[idx_v], xrow_v, sem).wait()
        pltpu.sync_copy(prow_v, outp_hbm.at[pl.ds(base, bpw)])
        pltpu.sync_copy(xrow_v, outx_hbm.at[pl.ds(base, bpw)])

    return k(ptab, xtab, idx)


def kernel(xyz, points):
    _x = jnp.squeeze(xyz, -1)
    xyzr = _x.reshape(B, 3, SUB, 128)
    xyzc3 = jnp.transpose(_x, (0, 2, 1))
    ptab = jnp.transpose(jnp.squeeze(points, -1), (0, 2, 1)).reshape(B * N, CP)
    xtab = jnp.pad(xyzc3, ((0, 0), (0, 0), (0, 125))).reshape(B * N, 128)

    gidx3 = _tc_indices(xyzr)
    gidx = gidx3.reshape(B * S)
    outp, outx = _sc_gather(ptab, xtab, gidx)
    new_xyz = outx.reshape(B, S, 128)[:, :, :3]
    new_points = outp.reshape(B, S, CP)
    return new_xyz, new_points


# speedup vs baseline: 7.7784x; 1.7026x over previous
import functools

import jax
import jax.numpy as jnp
from jax import lax
from jax.experimental import pallas as pl
from jax.experimental.pallas import tpu as pltpu
from jax.experimental.pallas import tpu_sc as plsc

B = 4
N = 2048
S = N // 2
CP = 128
SUB = 16
BLK = 128
MIN_R2 = 0.02 ** 2
MAX_R2 = 0.05 ** 2


def _r2max(x):
    return jnp.max(jnp.max(x, axis=2, keepdims=True), axis=1, keepdims=True)


def _r2min(x):
    return jnp.min(jnp.min(x, axis=2, keepdims=True), axis=1, keepdims=True)


def _r2sum(x):
    return jnp.sum(jnp.sum(x, axis=2, keepdims=True), axis=1, keepdims=True)


def _tc_body(xyzr_ref, xcols_ref, out_ref, cenx_s, ceny_s, cenz_s, fps_s):
    X = xyzr_ref[:, 0]
    Y = xyzr_ref[:, 1]
    Z = xyzr_ref[:, 2]
    iota3 = (lax.broadcasted_iota(jnp.int32, (B, SUB, 128), 1) * 128
             + lax.broadcasted_iota(jnp.int32, (B, SUB, 128), 2))
    lane = lax.broadcasted_iota(jnp.int32, (B, BLK), 1)

    def step(i, carry):
        D, far, bfar, bcx, bcy, bcz = carry
        sel = iota3 == far
        cx = _r2sum(jnp.where(sel, X, 0.0))
        cy = _r2sum(jnp.where(sel, Y, 0.0))
        cz = _r2sum(jnp.where(sel, Z, 0.0))
        dx = X - cx
        dy = Y - cy
        dz = Z - cz
        dist = (dx * dx + dy * dy) + dz * dz
        D = jnp.minimum(D, dist)
        m = _r2max(D)
        far_new = _r2min(jnp.where(D == m, iota3, N))
        put = lane == i
        bfar = jnp.where(put, far[:, :, 0], bfar)
        bcx = jnp.where(put, cx[:, :, 0], bcx)
        bcy = jnp.where(put, cy[:, :, 0], bcy)
        bcz = jnp.where(put, cz[:, :, 0], bcz)
        return D, far_new, bfar, bcx, bcy, bcz

    nblk = S // BLK

    def fps_blk(blk, carry):
        D, far = carry
        bfar = jnp.zeros((B, BLK), jnp.int32)
        bcx = jnp.zeros((B, BLK), jnp.float32)
        bcy = jnp.zeros((B, BLK), jnp.float32)
        bcz = jnp.zeros((B, BLK), jnp.float32)
        D, far, bfar, bcx, bcy, bcz = lax.fori_loop(
            0, BLK, step, (D, far, bfar, bcx, bcy, bcz))
        fps_s[:, pl.ds(blk, 1), :] = bfar.reshape(B, 1, BLK)
        cenx_s[:, pl.ds(blk, 1), :] = bcx.reshape(B, 1, BLK)
        ceny_s[:, pl.ds(blk, 1), :] = bcy.reshape(B, 1, BLK)
        cenz_s[:, pl.ds(blk, 1), :] = bcz.reshape(B, 1, BLK)
        return D, far

    D0 = jnp.full((B, SUB, 128), 1e10, jnp.float32)
    far0 = jnp.zeros((B, 1, 1), jnp.int32)
    lax.fori_loop(0, nblk, fps_blk, (D0, far0))

    iota_n = lax.broadcasted_iota(jnp.int32, (N, 1), 0)
    iota_blk = lax.broadcasted_iota(jnp.int32, (nblk, BLK), 0)

    for b in range(B):
        Xc = xcols_ref[b, 0]
        Yc = xcols_ref[b, 1]
        Zc = xcols_ref[b, 2]
        sn = (Xc * Xc + Yc * Yc) + Zc * Zc
        Xcb = Xc.astype(jnp.bfloat16).astype(jnp.float32)
        Ycb = Yc.astype(jnp.bfloat16).astype(jnp.float32)
        Zcb = Zc.astype(jnp.bfloat16).astype(jnp.float32)
        cenx_all = cenx_s[b]
        ceny_all = ceny_s[b]
        cenz_all = cenz_s[b]
        fps_all = fps_s[b]

        def bq(blk, _, b=b, Xcb=Xcb, Ycb=Ycb, Zcb=Zcb, sn=sn,
               cenx_all=cenx_all, ceny_all=ceny_all, cenz_all=cenz_all,
               fps_all=fps_all):
            pick = iota_blk == blk
            cx = jnp.sum(jnp.where(pick, cenx_all, 0.0), axis=0,
                         keepdims=True)
            cy = jnp.sum(jnp.where(pick, ceny_all, 0.0), axis=0,
                         keepdims=True)
            cz = jnp.sum(jnp.where(pick, cenz_all, 0.0), axis=0,
                         keepdims=True)
            sc = (cx * cx + cy * cy) + cz * cz
            cxb = cx.astype(jnp.bfloat16).astype(jnp.float32)
            cyb = cy.astype(jnp.bfloat16).astype(jnp.float32)
            czb = cz.astype(jnp.bfloat16).astype(jnp.float32)
            E = (Xcb * cxb + Ycb * cyb) + Zcb * czb
            d = (-2.0 * E + sc) + sn
            mask = (d > MIN_R2) & (d < MAX_R2)
            cand = jnp.where(mask, d, jnp.inf)
            mn = jnp.min(cand, axis=0, keepdims=True)
            near = jnp.min(jnp.where(cand == mn, iota_n, N), axis=0,
                           keepdims=True)
            has = mn < jnp.inf
            fpsr = jnp.sum(jnp.where(pick, fps_all, 0), axis=0, keepdims=True)
            row = jnp.where(has, near, fpsr) + b * N
            out_ref[b, pl.ds(blk, 1), :] = row
            return 0

        lax.fori_loop(0, nblk, bq, 0)


@jax.jit
def _tc_indices(xyzr, xcols):
    return pl.pallas_call(
        _tc_body,
        out_shape=jax.ShapeDtypeStruct((B, S // BLK, BLK), jnp.int32),
        scratch_shapes=[
            pltpu.VMEM((B, S // BLK, BLK), jnp.float32),
            pltpu.VMEM((B, S // BLK, BLK), jnp.float32),
            pltpu.VMEM((B, S // BLK, BLK), jnp.float32),
            pltpu.VMEM((B, S // BLK, BLK), jnp.int32),
        ],
    )(xyzr, xcols)


@jax.jit
def _sc_gather(ptab, xtab, idx):
    info = plsc.get_sparse_core_info()
    nc, ns = info.num_cores, info.num_subcores
    nw = nc * ns
    bpw = (B * S) // nw
    mesh = plsc.VectorSubcoreMesh(core_axis_name="c", subcore_axis_name="s")

    @functools.partial(
        pl.kernel,
        out_type=(jax.ShapeDtypeStruct((B * S, CP), jnp.float32),
                  jax.ShapeDtypeStruct((B * S, 128), jnp.float32)),
        mesh=mesh,
        scratch_types=[
            pltpu.VMEM((bpw,), jnp.int32),
            pltpu.VMEM((bpw, CP), jnp.float32),
            pltpu.VMEM((bpw, 128), jnp.float32),
            pltpu.SemaphoreType.DMA,
        ],
    )
    def k(ptab_hbm, xtab_hbm, idx_hbm, outp_hbm, outx_hbm,
          idx_v, prow_v, xrow_v, sem):
        wid = lax.axis_index("s") * nc + lax.axis_index("c")
        base = wid * bpw
        pltpu.sync_copy(idx_hbm.at[pl.ds(base, bpw)], idx_v)
        pltpu.async_copy(ptab_hbm.at[idx_v], prow_v, sem).wait()
        pltpu.async_copy(xtab_hbm.at[idx_v], xrow_v, sem).wait()
        pltpu.sync_copy(prow_v, outp_hbm.at[pl.ds(base, bpw)])
        pltpu.sync_copy(xrow_v, outx_hbm.at[pl.ds(base, bpw)])

    return k(ptab, xtab, idx)


def kernel(xyz, points):
    _x = jnp.squeeze(xyz, -1)
    xyzr = _x.reshape(B, 3, SUB, 128)
    xyzc3 = jnp.transpose(_x, (0, 2, 1))
    ptab = jnp.transpose(jnp.squeeze(points, -1), (0, 2, 1)).reshape(B * N, CP)
    xtab = jnp.pad(xyzc3, ((0, 0), (0, 0), (0, 125))).reshape(B * N, 128)

    xcols = _x[:, :, :, None]
    gidx3 = _tc_indices(xyzr, xcols)
    gidx = gidx3.reshape(B * S)
    outp, outx = _sc_gather(ptab, xtab, gidx)
    new_xyz = outx.reshape(B, S, 128)[:, :, :3]
    new_points = outp.reshape(B, S, CP)
    return new_xyz, new_points
